# Initial kernel scaffold; baseline (speedup 1.0000x reference)
import functools

import jax
import jax.numpy as jnp
from jax import lax
from jax.experimental import pallas as pl
from jax.experimental.pallas import tpu as pltpu
from jax.experimental.pallas import tpu_sc as plsc

N = 10000
E = 320000
D = 128

NC = 2
NS = 16
NW = NC * NS
EPT = E // NW
CH = 80
NCHUNK = EPT // CH
RPT = N // NS


@functools.lru_cache(maxsize=1)
def _build_sc_spmm():
    mesh = plsc.VectorSubcoreMesh(core_axis_name="c", subcore_axis_name="s")

    @functools.partial(
        pl.kernel,
        out_type=jax.ShapeDtypeStruct((NC, N, D), jnp.float32),
        mesh=mesh,
        scratch_types=[
            pltpu.VMEM((CH,), jnp.int32),
            pltpu.VMEM((CH,), jnp.int32),
            pltpu.VMEM((EPT,), jnp.float32),
            pltpu.VMEM((CH, D), jnp.float32),
            pltpu.VMEM_SHARED((N, D), jnp.float32),
            pltpu.SemaphoreType.DMA,
        ],
    )
    def sc_spmm(x_hbm, col_hbm, row_hbm, w_hbm, zeros_hbm, out_hbm,
                cidx, ridx, wbuf, rows, aggbuf, sem):
        c = lax.axis_index("c")
        s = lax.axis_index("s")
        wid = s * NC + c
        ebase = wid * EPT

        pltpu.sync_copy(zeros_hbm.at[pl.ds(s * RPT, RPT)],
                        aggbuf.at[pl.ds(s * RPT, RPT)])
        pltpu.sync_copy(w_hbm.at[pl.ds(ebase, EPT)], wbuf)
        plsc.subcore_barrier()

        def chunk_body(k, carry):
            base = ebase + k * CH
            pltpu.sync_copy(col_hbm.at[pl.ds(base, CH)], cidx)
            pltpu.sync_copy(row_hbm.at[pl.ds(base, CH)], ridx)
            pltpu.async_copy(x_hbm.at[cidx], rows, sem).wait()

            def scale_body(i, carry2):
                w = wbuf[k * CH + i]
                for j in range(D // 16):
                    sl = pl.ds(j * 16, 16)
                    rows[i, sl] = rows[i, sl] * w
                return carry2

            lax.fori_loop(0, CH, scale_body, 0)
            pltpu.sync_copy(rows, aggbuf.at[ridx], add=True)
            return carry

        lax.fori_loop(0, NCHUNK, chunk_body, 0)
        plsc.subcore_barrier()
        pltpu.sync_copy(aggbuf.at[pl.ds(s * RPT, RPT)],
                        out_hbm.at[c, pl.ds(s * RPT, RPT)])

    return sc_spmm


BLK = 1000


def _tc_feat_kernel(x_ref, p_ref, w0_ref, w1_ref, b0_ref, b1_ref,
                    feat_ref, s_ref, ss_ref):
    i = pl.program_id(0)
    xb = x_ref[...]
    aggb = p_ref[0] + p_ref[1]
    h0 = jnp.maximum(
        jnp.dot(xb, w0_ref[...], preferred_element_type=jnp.float32)
        + b0_ref[...], 0.0)
    h1 = jnp.maximum(
        jnp.dot(aggb, w1_ref[...], preferred_element_type=jnp.float32)
        + b1_ref[...], 0.0)
    f = h0 + h1
    feat_ref[...] = f
    sb = jnp.sum(f, axis=0, keepdims=True)
    ssb = jnp.sum(f * f, axis=0, keepdims=True)

    @pl.when(i == 0)
    def _():
        s_ref[...] = sb
        ss_ref[...] = ssb

    @pl.when(i != 0)
    def _():
        s_ref[...] += sb
        ss_ref[...] += ssb


def _tc_bn_kernel(feat_ref, s_ref, ss_ref, g_ref, bt_ref, out_ref):
    mean = s_ref[...] / N
    var = ss_ref[...] / N - mean * mean
    scale = lax.rsqrt(var + 1e-9) * g_ref[...]
    out_ref[...] = feat_ref[...] * scale + (bt_ref[...] - mean * scale)


def kernel(x, edge_index, edge_weight, W0, W1, b0, b1, gamma, beta):
    row = edge_index[0]
    col = edge_index[1]
    zeros = jnp.zeros((N, D), jnp.float32)

    part = _build_sc_spmm()(x, col, row, edge_weight, zeros)

    feat, s, ss = pl.pallas_call(
        _tc_feat_kernel,
        grid=(N // BLK,),
        in_specs=[
            pl.BlockSpec((BLK, D), lambda i: (i, 0)),
            pl.BlockSpec((NC, BLK, D), lambda i: (0, i, 0)),
            pl.BlockSpec((D, D), lambda i: (0, 0)),
            pl.BlockSpec((D, D), lambda i: (0, 0)),
            pl.BlockSpec((1, D), lambda i: (0, 0)),
            pl.BlockSpec((1, D), lambda i: (0, 0)),
        ],
        out_specs=[
            pl.BlockSpec((BLK, D), lambda i: (i, 0)),
            pl.BlockSpec((1, D), lambda i: (0, 0)),
            pl.BlockSpec((1, D), lambda i: (0, 0)),
        ],
        out_shape=[
            jax.ShapeDtypeStruct((N, D), jnp.float32),
            jax.ShapeDtypeStruct((1, D), jnp.float32),
            jax.ShapeDtypeStruct((1, D), jnp.float32),
        ],
    )(x, part, W0, W1, b0[None, :], b1[None, :])

    out = pl.pallas_call(
        _tc_bn_kernel,
        grid=(N // BLK,),
        in_specs=[
            pl.BlockSpec((BLK, D), lambda i: (i, 0)),
            pl.BlockSpec((1, D), lambda i: (0, 0)),
            pl.BlockSpec((1, D), lambda i: (0, 0)),
            pl.BlockSpec((1, D), lambda i: (0, 0)),
            pl.BlockSpec((1, D), lambda i: (0, 0)),
        ],
        out_specs=pl.BlockSpec((BLK, D), lambda i: (i, 0)),
        out_shape=jax.ShapeDtypeStruct((N, D), jnp.float32),
    )(feat, s, ss, gamma[None, :], beta[None, :])
    return out


# speedup vs baseline: 4.9480x; 4.9480x over previous
import functools

import jax
import jax.numpy as jnp
from jax import lax
from jax.experimental import pallas as pl
from jax.experimental.pallas import tpu as pltpu
from jax.experimental.pallas import tpu_sc as plsc

N = 10000
E = 320000
D = 128

NC = 2
NS = 16
NW = NC * NS
EPT = E // NW
CH = 80
NCHUNK = EPT // CH
ZR = 624
ZR_LAST = N - (NS - 1) * ZR


@functools.lru_cache(maxsize=1)
def _build_sc_spmm():
    mesh = plsc.VectorSubcoreMesh(core_axis_name="c", subcore_axis_name="s")

    @functools.partial(
        pl.kernel,
        out_type=jax.ShapeDtypeStruct((NC, N, D), jnp.float32),
        mesh=mesh,
        scratch_types=[
            pltpu.VMEM((CH,), jnp.int32),
            pltpu.VMEM((CH,), jnp.int32),
            pltpu.VMEM((EPT,), jnp.float32),
            pltpu.VMEM((CH, D), jnp.float32),
            pltpu.VMEM_SHARED((N, D), jnp.float32),
            pltpu.SemaphoreType.DMA,
        ],
    )
    def sc_spmm(x_hbm, col_hbm, row_hbm, w_hbm, zeros_hbm, out_hbm,
                cidx, ridx, wbuf, rows, aggbuf, sem):
        c = lax.axis_index("c")
        s = lax.axis_index("s")
        wid = s * NC + c
        ebase = wid * EPT

        sbase = pl.multiple_of(s * ZR, 8)

        @pl.when(s < NS - 1)
        def _():
            pltpu.sync_copy(zeros_hbm.at[pl.ds(sbase, ZR)],
                            aggbuf.at[pl.ds(sbase, ZR)])

        @pl.when(s == NS - 1)
        def _():
            pltpu.sync_copy(zeros_hbm.at[pl.ds((NS - 1) * ZR, ZR_LAST)],
                            aggbuf.at[pl.ds((NS - 1) * ZR, ZR_LAST)])
        pltpu.sync_copy(w_hbm.at[pl.ds(ebase, EPT)], wbuf)
        plsc.subcore_barrier()

        def chunk_body(k, carry):
            base = ebase + k * CH
            pltpu.sync_copy(col_hbm.at[pl.ds(base, CH)], cidx)
            pltpu.sync_copy(row_hbm.at[pl.ds(base, CH)], ridx)
            pltpu.async_copy(x_hbm.at[cidx], rows, sem).wait()

            def scale_body(i16, carry2):
                wv = wbuf[pl.ds(k * CH + i16 * 16, 16)]
                for l in range(16):
                    w = wv[l]
                    for j in range(D // 16):
                        sl = pl.ds(j * 16, 16)
                        rows[i16 * 16 + l, sl] = rows[i16 * 16 + l, sl] * w
                return carry2

            lax.fori_loop(0, CH // 16, scale_body, 0)
            pltpu.sync_copy(rows, aggbuf.at[ridx], add=True)
            return carry

        lax.fori_loop(0, NCHUNK, chunk_body, 0)
        plsc.subcore_barrier()

        @pl.when(s < NS - 1)
        def _():
            pltpu.sync_copy(aggbuf.at[pl.ds(sbase, ZR)],
                            out_hbm.at[c, pl.ds(sbase, ZR)])

        @pl.when(s == NS - 1)
        def _():
            pltpu.sync_copy(aggbuf.at[pl.ds((NS - 1) * ZR, ZR_LAST)],
                            out_hbm.at[c, pl.ds((NS - 1) * ZR, ZR_LAST)])

    return sc_spmm


BLK = 1000


def _tc_feat_kernel(x_ref, p_ref, w0_ref, w1_ref, b0_ref, b1_ref,
                    feat_ref, s_ref, ss_ref):
    i = pl.program_id(0)
    xb = x_ref[...]
    aggb = p_ref[0] + p_ref[1]
    h0 = jnp.maximum(
        jnp.dot(xb, w0_ref[...], preferred_element_type=jnp.float32)
        + b0_ref[...], 0.0)
    h1 = jnp.maximum(
        jnp.dot(aggb, w1_ref[...], preferred_element_type=jnp.float32)
        + b1_ref[...], 0.0)
    f = h0 + h1
    feat_ref[...] = f
    sb = jnp.sum(f, axis=0, keepdims=True)
    ssb = jnp.sum(f * f, axis=0, keepdims=True)

    @pl.when(i == 0)
    def _():
        s_ref[...] = sb
        ss_ref[...] = ssb

    @pl.when(i != 0)
    def _():
        s_ref[...] += sb
        ss_ref[...] += ssb


def _tc_bn_kernel(feat_ref, s_ref, ss_ref, g_ref, bt_ref, out_ref):
    mean = s_ref[...] / N
    var = ss_ref[...] / N - mean * mean
    scale = lax.rsqrt(var + 1e-9) * g_ref[...]
    out_ref[...] = feat_ref[...] * scale + (bt_ref[...] - mean * scale)


def kernel(x, edge_index, edge_weight, W0, W1, b0, b1, gamma, beta):
    row = edge_index[0]
    col = edge_index[1]
    zeros = jnp.zeros((N, D), jnp.float32)

    part = _build_sc_spmm()(x, col, row, edge_weight, zeros)

    feat, s, ss = pl.pallas_call(
        _tc_feat_kernel,
        grid=(N // BLK,),
        in_specs=[
            pl.BlockSpec((BLK, D), lambda i: (i, 0)),
            pl.BlockSpec((NC, BLK, D), lambda i: (0, i, 0)),
            pl.BlockSpec((D, D), lambda i: (0, 0)),
            pl.BlockSpec((D, D), lambda i: (0, 0)),
            pl.BlockSpec((1, D), lambda i: (0, 0)),
            pl.BlockSpec((1, D), lambda i: (0, 0)),
        ],
        out_specs=[
            pl.BlockSpec((BLK, D), lambda i: (i, 0)),
            pl.BlockSpec((1, D), lambda i: (0, 0)),
            pl.BlockSpec((1, D), lambda i: (0, 0)),
        ],
        out_shape=[
            jax.ShapeDtypeStruct((N, D), jnp.float32),
            jax.ShapeDtypeStruct((1, D), jnp.float32),
            jax.ShapeDtypeStruct((1, D), jnp.float32),
        ],
    )(x, part, W0, W1, b0[None, :], b1[None, :])

    out = pl.pallas_call(
        _tc_bn_kernel,
        grid=(N // BLK,),
        in_specs=[
            pl.BlockSpec((BLK, D), lambda i: (i, 0)),
            pl.BlockSpec((1, D), lambda i: (0, 0)),
            pl.BlockSpec((1, D), lambda i: (0, 0)),
            pl.BlockSpec((1, D), lambda i: (0, 0)),
            pl.BlockSpec((1, D), lambda i: (0, 0)),
        ],
        out_specs=pl.BlockSpec((BLK, D), lambda i: (i, 0)),
        out_shape=jax.ShapeDtypeStruct((N, D), jnp.float32),
    )(feat, s, ss, gamma[None, :], beta[None, :])
    return out


# speedup vs baseline: 7.7557x; 1.5674x over previous
import functools

import jax
import jax.numpy as jnp
from jax import lax
from jax.experimental import pallas as pl
from jax.experimental.pallas import tpu as pltpu
from jax.experimental.pallas import tpu_sc as plsc

N = 10000
E = 320000
D = 128

NC = 2
NS = 16
NW = NC * NS
EPT = E // NW
CH = 80
NCHUNK = EPT // CH
ZR = 624
ZR_LAST = N - (NS - 1) * ZR


@functools.lru_cache(maxsize=1)
def _build_sc_spmm():
    mesh = plsc.VectorSubcoreMesh(core_axis_name="c", subcore_axis_name="s")

    @functools.partial(
        pl.kernel,
        out_type=jax.ShapeDtypeStruct((NC, N, D), jnp.float32),
        mesh=mesh,
        scratch_types=[
            pltpu.VMEM((EPT,), jnp.int32),
            pltpu.VMEM((EPT,), jnp.float32),
            pltpu.VMEM((CH, D), jnp.float32),
            pltpu.VMEM((CH, D), jnp.float32),
            pltpu.VMEM_SHARED((N, D), jnp.float32),
            pltpu.SemaphoreType.DMA,
            pltpu.SemaphoreType.DMA,
            pltpu.SemaphoreType.DMA,
            pltpu.SemaphoreType.DMA,
        ],
    )
    def sc_spmm(x_hbm, packed_hbm, w_hbm, zeros_hbm, out_hbm,
                pall, wbuf, b0, b1, aggbuf,
                gsem0, gsem1, ssem0, ssem1):
        c = lax.axis_index("c")
        s = lax.axis_index("s")
        wid = s * NC + c
        ebase = wid * EPT

        sbase = pl.multiple_of(s * ZR, 8)

        @pl.when(s < NS - 1)
        def _():
            pltpu.sync_copy(zeros_hbm.at[pl.ds(sbase, ZR)],
                            aggbuf.at[pl.ds(sbase, ZR)])

        @pl.when(s == NS - 1)
        def _():
            pltpu.sync_copy(zeros_hbm.at[pl.ds((NS - 1) * ZR, ZR_LAST)],
                            aggbuf.at[pl.ds((NS - 1) * ZR, ZR_LAST)])
        pltpu.sync_copy(packed_hbm.at[pl.ds(ebase, EPT)], pall)
        pltpu.sync_copy(w_hbm.at[pl.ds(ebase, EPT)], wbuf)
        plsc.subcore_barrier()

        NG = CH // 16

        def src_idx(k, g):
            pk = pall[pl.ds(k * CH + g * 16, 16)]
            return jnp.bitwise_and(pk, 0xFFFF)

        def dst_idx(k, g):
            pk = pall[pl.ds(k * CH + g * 16, 16)]
            return jnp.right_shift(pk, 16)

        def launch_gather(k, buf, sem):
            for g in range(NG):
                pltpu.async_copy(x_hbm.at[src_idx(k, g)],
                                 buf.at[pl.ds(g * 16, 16)], sem)

        def wait_gather(k, buf, sem):
            for g in range(NG):
                pltpu.make_async_copy(x_hbm.at[src_idx(k, g)],
                                      buf.at[pl.ds(g * 16, 16)], sem).wait()

        def launch_scatter(k, buf, sem):
            for g in range(NG):
                pltpu.async_copy(buf.at[pl.ds(g * 16, 16)],
                                 aggbuf.at[dst_idx(k, g)], sem, add=True)

        def wait_scatter(k, buf, sem):
            for g in range(NG):
                pltpu.make_async_copy(buf.at[pl.ds(g * 16, 16)],
                                      aggbuf.at[dst_idx(k, g)], sem).wait()

        def scale(rows_ref, k):
            def scale_body(i16, carry2):
                wv = wbuf[pl.ds(k * CH + i16 * 16, 16)]
                for l in range(16):
                    w = wv[l]
                    for j in range(D // 16):
                        sl = pl.ds(j * 16, 16)
                        rows_ref[i16 * 16 + l, sl] = \
                            rows_ref[i16 * 16 + l, sl] * w
                return carry2

            lax.fori_loop(0, CH // 16, scale_body, 0)

        launch_gather(0, b0, gsem0)

        def chunk_body(k, carry):
            even = lax.rem(k, 2) == 0

            @pl.when(even)
            def _():
                wait_gather(k, b0, gsem0)
                scale(b0, k)
                launch_scatter(k, b0, ssem0)

                @pl.when(k + 1 < NCHUNK)
                def _():
                    @pl.when(k >= 1)
                    def _():
                        wait_scatter(k - 1, b1, ssem1)

                    launch_gather(k + 1, b1, gsem1)

            @pl.when(jnp.logical_not(even))
            def _():
                wait_gather(k, b1, gsem1)
                scale(b1, k)
                launch_scatter(k, b1, ssem1)

                @pl.when(k + 1 < NCHUNK)
                def _():
                    wait_scatter(k - 1, b0, ssem0)
                    launch_gather(k + 1, b0, gsem0)

            return carry

        lax.fori_loop(0, NCHUNK, chunk_body, 0)
        wait_scatter(NCHUNK - 2, b1, ssem1)
        wait_scatter(NCHUNK - 1, b0, ssem0)
        plsc.subcore_barrier()

        @pl.when(s < NS - 1)
        def _():
            pltpu.sync_copy(aggbuf.at[pl.ds(sbase, ZR)],
                            out_hbm.at[c, pl.ds(sbase, ZR)])

        @pl.when(s == NS - 1)
        def _():
            pltpu.sync_copy(aggbuf.at[pl.ds((NS - 1) * ZR, ZR_LAST)],
                            out_hbm.at[c, pl.ds((NS - 1) * ZR, ZR_LAST)])

    return sc_spmm


BLK = 1000


def _tc_feat_kernel(x_ref, p_ref, w0_ref, w1_ref, b0_ref, b1_ref,
                    feat_ref, s_ref, ss_ref):
    i = pl.program_id(0)
    xb = x_ref[...]
    aggb = p_ref[0] + p_ref[1]
    h0 = jnp.maximum(
        jnp.dot(xb, w0_ref[...], preferred_element_type=jnp.float32)
        + b0_ref[...], 0.0)
    h1 = jnp.maximum(
        jnp.dot(aggb, w1_ref[...], preferred_element_type=jnp.float32)
        + b1_ref[...], 0.0)
    f = h0 + h1
    feat_ref[...] = f
    sb = jnp.sum(f, axis=0, keepdims=True)
    ssb = jnp.sum(f * f, axis=0, keepdims=True)

    @pl.when(i == 0)
    def _():
        s_ref[...] = sb
        ss_ref[...] = ssb

    @pl.when(i != 0)
    def _():
        s_ref[...] += sb
        ss_ref[...] += ssb


def _tc_bn_kernel(feat_ref, s_ref, ss_ref, g_ref, bt_ref, out_ref):
    mean = s_ref[...] / N
    var = ss_ref[...] / N - mean * mean
    scale = lax.rsqrt(var + 1e-9) * g_ref[...]
    out_ref[...] = feat_ref[...] * scale + (bt_ref[...] - mean * scale)


def kernel(x, edge_index, edge_weight, W0, W1, b0, b1, gamma, beta):
    packed = jnp.bitwise_or(jnp.left_shift(edge_index[0], 16), edge_index[1])
    zeros = jnp.zeros((N, D), jnp.float32)

    part = _build_sc_spmm()(x, packed, edge_weight, zeros)

    feat, s, ss = pl.pallas_call(
        _tc_feat_kernel,
        grid=(N // BLK,),
        in_specs=[
            pl.BlockSpec((BLK, D), lambda i: (i, 0)),
            pl.BlockSpec((NC, BLK, D), lambda i: (0, i, 0)),
            pl.BlockSpec((D, D), lambda i: (0, 0)),
            pl.BlockSpec((D, D), lambda i: (0, 0)),
            pl.BlockSpec((1, D), lambda i: (0, 0)),
            pl.BlockSpec((1, D), lambda i: (0, 0)),
        ],
        out_specs=[
            pl.BlockSpec((BLK, D), lambda i: (i, 0)),
            pl.BlockSpec((1, D), lambda i: (0, 0)),
            pl.BlockSpec((1, D), lambda i: (0, 0)),
        ],
        out_shape=[
            jax.ShapeDtypeStruct((N, D), jnp.float32),
            jax.ShapeDtypeStruct((1, D), jnp.float32),
            jax.ShapeDtypeStruct((1, D), jnp.float32),
        ],
    )(x, part, W0, W1, b0[None, :], b1[None, :])

    out = pl.pallas_call(
        _tc_bn_kernel,
        grid=(N // BLK,),
        in_specs=[
            pl.BlockSpec((BLK, D), lambda i: (i, 0)),
            pl.BlockSpec((1, D), lambda i: (0, 0)),
            pl.BlockSpec((1, D), lambda i: (0, 0)),
            pl.BlockSpec((1, D), lambda i: (0, 0)),
            pl.BlockSpec((1, D), lambda i: (0, 0)),
        ],
        out_specs=pl.BlockSpec((BLK, D), lambda i: (i, 0)),
        out_shape=jax.ShapeDtypeStruct((N, D), jnp.float32),
    )(feat, s, ss, gamma[None, :], beta[None, :])
    return out


# speedup vs baseline: 9.8320x; 1.2677x over previous
import functools

import jax
import jax.numpy as jnp
from jax import lax
from jax.experimental import pallas as pl
from jax.experimental.pallas import tpu as pltpu
from jax.experimental.pallas import tpu_sc as plsc

N = 10000
E = 320000
D = 128

NC = 2
NS = 16
NW = NC * NS
EPT = E // NW
CH = 80
NCHUNK = EPT // CH
ZR = 624
ZR_LAST = N - (NS - 1) * ZR


@functools.lru_cache(maxsize=1)
def _build_sc_spmm():
    mesh = plsc.VectorSubcoreMesh(core_axis_name="c", subcore_axis_name="s")

    @functools.partial(
        pl.kernel,
        out_type=jax.ShapeDtypeStruct((NC, N, D), jnp.float32),
        mesh=mesh,
        scratch_types=[
            pltpu.VMEM((EPT,), jnp.int32),
            pltpu.VMEM((EPT,), jnp.float32),
            pltpu.VMEM((CH, D), jnp.float32),
            pltpu.VMEM((CH, D), jnp.float32),
            pltpu.VMEM((CH, D), jnp.float32),
            pltpu.VMEM_SHARED((N, D), jnp.float32),
            pltpu.SemaphoreType.DMA,
            pltpu.SemaphoreType.DMA,
            pltpu.SemaphoreType.DMA,
            pltpu.SemaphoreType.DMA,
            pltpu.SemaphoreType.DMA,
            pltpu.SemaphoreType.DMA,
        ],
    )
    def sc_spmm(x_hbm, packed_hbm, w_hbm, out_hbm,
                pall, wbuf, b0, b1, b2, aggbuf,
                gsem0, gsem1, gsem2, ssem0, ssem1, ssem2):
        c = lax.axis_index("c")
        s = lax.axis_index("s")
        wid = s * NC + c
        ebase = wid * EPT
        sbase = pl.multiple_of(s * ZR, 8)

        pltpu.sync_copy(packed_hbm.at[pl.ds(ebase, EPT)], pall)
        pltpu.sync_copy(w_hbm.at[pl.ds(ebase, EPT)], wbuf)

        zv = jnp.zeros((16,), jnp.float32)

        def zero_body(i, carry):
            for j in range(D // 16):
                b0[i, pl.ds(j * 16, 16)] = zv
            return carry

        lax.fori_loop(0, CH, zero_body, 0)
        for m in range(ZR // CH):
            pltpu.async_copy(b0, aggbuf.at[pl.ds(sbase + m * CH, CH)], gsem0)
        for m in range(ZR // CH):
            pltpu.make_async_copy(
                b0, aggbuf.at[pl.ds(sbase + m * CH, CH)], gsem0).wait()
        ZT = ZR - (ZR // CH) * CH

        @pl.when(s < NS - 1)
        def _():
            pltpu.sync_copy(b0.at[pl.ds(0, ZT)],
                            aggbuf.at[pl.ds(sbase + (ZR // CH) * CH, ZT)])

        @pl.when(s == NS - 1)
        def _():
            pltpu.sync_copy(b0,
                            aggbuf.at[pl.ds((NS - 1) * ZR + (ZR // CH) * CH,
                                            CH)])
        plsc.subcore_barrier()

        NG = CH // 16

        def src_idx(k, g):
            pk = pall[pl.ds(k * CH + g * 16, 16)]
            return jnp.bitwise_and(pk, 0xFFFF)

        def dst_idx(k, g):
            pk = pall[pl.ds(k * CH + g * 16, 16)]
            return jnp.right_shift(pk, 16)

        def launch_gather(k, buf, sem):
            for g in range(NG):
                pltpu.async_copy(x_hbm.at[src_idx(k, g)],
                                 buf.at[pl.ds(g * 16, 16)], sem)

        def wait_gather(k, buf, sem):
            for g in range(NG):
                pltpu.make_async_copy(x_hbm.at[src_idx(k, g)],
                                      buf.at[pl.ds(g * 16, 16)], sem).wait()

        def launch_scatter(k, buf, sem):
            for g in range(NG):
                pltpu.async_copy(buf.at[pl.ds(g * 16, 16)],
                                 aggbuf.at[dst_idx(k, g)], sem, add=True)

        def wait_scatter(k, buf, sem):
            for g in range(NG):
                pltpu.make_async_copy(buf.at[pl.ds(g * 16, 16)],
                                      aggbuf.at[dst_idx(k, g)], sem).wait()

        def scale(rows_ref, k):
            def scale_body(i16, carry2):
                wv = wbuf[pl.ds(k * CH + i16 * 16, 16)]
                for l in range(16):
                    w = wv[l]
                    for j in range(D // 16):
                        sl = pl.ds(j * 16, 16)
                        rows_ref[i16 * 16 + l, sl] = \
                            rows_ref[i16 * 16 + l, sl] * w
                return carry2

            lax.fori_loop(0, CH // 16, scale_body, 0)

        bufs = (b0, b1, b2)
        gsems = (gsem0, gsem1, gsem2)
        ssems = (ssem0, ssem1, ssem2)

        launch_gather(0, b0, gsem0)

        def chunk_body(k, carry):
            m = lax.rem(k, 3)
            for b in range(3):
                nb = (b + 1) % 3

                @pl.when(m == b)
                def _(b=b, nb=nb):
                    wait_gather(k, bufs[b], gsems[b])

                    @pl.when(k + 1 < NCHUNK)
                    def _():
                        @pl.when(k >= 2)
                        def _():
                            wait_scatter(k - 2, bufs[nb], ssems[nb])

                        launch_gather(k + 1, bufs[nb], gsems[nb])

                    scale(bufs[b], k)
                    launch_scatter(k, bufs[b], ssems[b])

            return carry

        lax.fori_loop(0, NCHUNK, chunk_body, 0)
        for kk in (NCHUNK - 3, NCHUNK - 2, NCHUNK - 1):
            wait_scatter(kk, bufs[kk % 3], ssems[kk % 3])
        plsc.subcore_barrier()

        @pl.when(s < NS - 1)
        def _():
            pltpu.sync_copy(aggbuf.at[pl.ds(sbase, ZR)],
                            out_hbm.at[c, pl.ds(sbase, ZR)])

        @pl.when(s == NS - 1)
        def _():
            pltpu.sync_copy(aggbuf.at[pl.ds((NS - 1) * ZR, ZR_LAST)],
                            out_hbm.at[c, pl.ds((NS - 1) * ZR, ZR_LAST)])

    return sc_spmm


BLK = 1000


def _tc_feat_kernel(x_ref, p_ref, w0_ref, w1_ref, b0_ref, b1_ref,
                    feat_ref, s_ref, ss_ref):
    i = pl.program_id(0)
    xb = x_ref[...]
    aggb = p_ref[0] + p_ref[1]
    h0 = jnp.maximum(
        jnp.dot(xb, w0_ref[...], preferred_element_type=jnp.float32)
        + b0_ref[...], 0.0)
    h1 = jnp.maximum(
        jnp.dot(aggb, w1_ref[...], preferred_element_type=jnp.float32)
        + b1_ref[...], 0.0)
    f = h0 + h1
    feat_ref[...] = f
    sb = jnp.sum(f, axis=0, keepdims=True)
    ssb = jnp.sum(f * f, axis=0, keepdims=True)

    @pl.when(i == 0)
    def _():
        s_ref[...] = sb
        ss_ref[...] = ssb

    @pl.when(i != 0)
    def _():
        s_ref[...] += sb
        ss_ref[...] += ssb


def _tc_bn_kernel(feat_ref, s_ref, ss_ref, g_ref, bt_ref, out_ref):
    mean = s_ref[...] / N
    var = ss_ref[...] / N - mean * mean
    scale = lax.rsqrt(var + 1e-9) * g_ref[...]
    out_ref[...] = feat_ref[...] * scale + (bt_ref[...] - mean * scale)


def kernel(x, edge_index, edge_weight, W0, W1, b0, b1, gamma, beta):
    packed = jnp.bitwise_or(jnp.left_shift(edge_index[0], 16), edge_index[1])

    part = _build_sc_spmm()(x, packed, edge_weight)

    feat, s, ss = pl.pallas_call(
        _tc_feat_kernel,
        grid=(N // BLK,),
        in_specs=[
            pl.BlockSpec((BLK, D), lambda i: (i, 0)),
            pl.BlockSpec((NC, BLK, D), lambda i: (0, i, 0)),
            pl.BlockSpec((D, D), lambda i: (0, 0)),
            pl.BlockSpec((D, D), lambda i: (0, 0)),
            pl.BlockSpec((1, D), lambda i: (0, 0)),
            pl.BlockSpec((1, D), lambda i: (0, 0)),
        ],
        out_specs=[
            pl.BlockSpec((BLK, D), lambda i: (i, 0)),
            pl.BlockSpec((1, D), lambda i: (0, 0)),
            pl.BlockSpec((1, D), lambda i: (0, 0)),
        ],
        out_shape=[
            jax.ShapeDtypeStruct((N, D), jnp.float32),
            jax.ShapeDtypeStruct((1, D), jnp.float32),
            jax.ShapeDtypeStruct((1, D), jnp.float32),
        ],
    )(x, part, W0, W1, b0[None, :], b1[None, :])

    out = pl.pallas_call(
        _tc_bn_kernel,
        grid=(N // BLK,),
        in_specs=[
            pl.BlockSpec((BLK, D), lambda i: (i, 0)),
            pl.BlockSpec((1, D), lambda i: (0, 0)),
            pl.BlockSpec((1, D), lambda i: (0, 0)),
            pl.BlockSpec((1, D), lambda i: (0, 0)),
            pl.BlockSpec((1, D), lambda i: (0, 0)),
        ],
        out_specs=pl.BlockSpec((BLK, D), lambda i: (i, 0)),
        out_shape=jax.ShapeDtypeStruct((N, D), jnp.float32),
    )(feat, s, ss, gamma[None, :], beta[None, :])
    return out


# speedup vs baseline: 10.1107x; 1.0283x over previous
import functools

import jax
import jax.numpy as jnp
from jax import lax
from jax.experimental import pallas as pl
from jax.experimental.pallas import tpu as pltpu
from jax.experimental.pallas import tpu_sc as plsc

N = 10000
E = 320000
D = 128

NC = 2
NS = 16
NW = NC * NS
EPT = E // NW
CH = 80
NCHUNK = EPT // CH
ZR = 624
ZR_LAST = N - (NS - 1) * ZR


@functools.lru_cache(maxsize=1)
def _build_sc_spmm():
    mesh = plsc.VectorSubcoreMesh(core_axis_name="c", subcore_axis_name="s")

    @functools.partial(
        pl.kernel,
        out_type=jax.ShapeDtypeStruct((NC, N, D), jnp.float32),
        mesh=mesh,
        scratch_types=[
            pltpu.VMEM((EPT,), jnp.int32),
            pltpu.VMEM((EPT,), jnp.float32),
            pltpu.VMEM((CH, D), jnp.float32),
            pltpu.VMEM((CH, D), jnp.float32),
            pltpu.VMEM((CH, D), jnp.float32),
            pltpu.VMEM_SHARED((N, D), jnp.float32),
            pltpu.SemaphoreType.DMA,
            pltpu.SemaphoreType.DMA,
            pltpu.SemaphoreType.DMA,
            pltpu.SemaphoreType.DMA,
            pltpu.SemaphoreType.DMA,
            pltpu.SemaphoreType.DMA,
        ],
    )
    def sc_spmm(x_hbm, packed_hbm, w_hbm, out_hbm,
                pall, wbuf, b0, b1, b2, aggbuf,
                gsem0, gsem1, gsem2, ssem0, ssem1, ssem2):
        c = lax.axis_index("c")
        s = lax.axis_index("s")
        wid = s * NC + c
        ebase = wid * EPT
        sbase = pl.multiple_of(s * ZR, 8)

        pltpu.sync_copy(packed_hbm.at[pl.ds(ebase, EPT)], pall)
        pltpu.sync_copy(w_hbm.at[pl.ds(ebase, EPT)], wbuf)

        zv = jnp.zeros((16,), jnp.float32)

        def zero_body(i, carry):
            for j in range(D // 16):
                b0[i, pl.ds(j * 16, 16)] = zv
            return carry

        lax.fori_loop(0, CH, zero_body, 0)
        for m in range(ZR // CH):
            pltpu.async_copy(b0, aggbuf.at[pl.ds(sbase + m * CH, CH)], gsem0)
        for m in range(ZR // CH):
            pltpu.make_async_copy(
                b0, aggbuf.at[pl.ds(sbase + m * CH, CH)], gsem0).wait()
        ZT = ZR - (ZR // CH) * CH

        @pl.when(s < NS - 1)
        def _():
            pltpu.sync_copy(b0.at[pl.ds(0, ZT)],
                            aggbuf.at[pl.ds(sbase + (ZR // CH) * CH, ZT)])

        @pl.when(s == NS - 1)
        def _():
            pltpu.sync_copy(b0,
                            aggbuf.at[pl.ds((NS - 1) * ZR + (ZR // CH) * CH,
                                            CH)])
        plsc.subcore_barrier()

        NG = CH // 16

        def src_idx(k, g):
            pk = pall[pl.ds(k * CH + g * 16, 16)]
            return jnp.bitwise_and(pk, 0xFFFF)

        def dst_idx(k, g):
            pk = pall[pl.ds(k * CH + g * 16, 16)]
            return jnp.right_shift(pk, 16)

        def launch_gather(k, buf, sem):
            for g in range(NG):
                pltpu.async_copy(x_hbm.at[src_idx(k, g)],
                                 buf.at[pl.ds(g * 16, 16)], sem)

        def wait_gather(k, buf, sem):
            for g in range(NG):
                pltpu.make_async_copy(x_hbm.at[src_idx(k, g)],
                                      buf.at[pl.ds(g * 16, 16)], sem).wait()

        def launch_scatter(k, buf, sem):
            for g in range(NG):
                pltpu.async_copy(buf.at[pl.ds(g * 16, 16)],
                                 aggbuf.at[dst_idx(k, g)], sem, add=True)

        def wait_scatter(k, buf, sem):
            for g in range(NG):
                pltpu.make_async_copy(buf.at[pl.ds(g * 16, 16)],
                                      aggbuf.at[dst_idx(k, g)], sem).wait()

        def scale(rows_ref, k):
            def scale_body(i16, carry2):
                wv = wbuf[pl.ds(k * CH + i16 * 16, 16)]
                for l in range(16):
                    w = wv[l]
                    for j in range(D // 16):
                        sl = pl.ds(j * 16, 16)
                        rows_ref[i16 * 16 + l, sl] = \
                            rows_ref[i16 * 16 + l, sl] * w
                return carry2

            lax.fori_loop(0, CH // 16, scale_body, 0)

        bufs = (b0, b1, b2)
        gsems = (gsem0, gsem1, gsem2)
        ssems = (ssem0, ssem1, ssem2)

        launch_gather(0, b0, gsem0)

        def chunk_body(k, carry):
            m = lax.rem(k, 3)
            for b in range(3):
                nb = (b + 1) % 3

                @pl.when(m == b)
                def _(b=b, nb=nb):
                    wait_gather(k, bufs[b], gsems[b])

                    @pl.when(k + 1 < NCHUNK)
                    def _():
                        @pl.when(k >= 2)
                        def _():
                            wait_scatter(k - 2, bufs[nb], ssems[nb])

                        launch_gather(k + 1, bufs[nb], gsems[nb])

                    scale(bufs[b], k)
                    launch_scatter(k, bufs[b], ssems[b])

            return carry

        lax.fori_loop(0, NCHUNK, chunk_body, 0)
        for kk in (NCHUNK - 3, NCHUNK - 2, NCHUNK - 1):
            wait_scatter(kk, bufs[kk % 3], ssems[kk % 3])
        plsc.subcore_barrier()

        @pl.when(s < NS - 1)
        def _():
            pltpu.sync_copy(aggbuf.at[pl.ds(sbase, ZR)],
                            out_hbm.at[c, pl.ds(sbase, ZR)])

        @pl.when(s == NS - 1)
        def _():
            pltpu.sync_copy(aggbuf.at[pl.ds((NS - 1) * ZR, ZR_LAST)],
                            out_hbm.at[c, pl.ds((NS - 1) * ZR, ZR_LAST)])

    return sc_spmm


BLK = 1000


def _tc_fused_kernel(x_ref, p_ref, w0_ref, w1_ref, b0_ref, b1_ref,
                     g_ref, bt_ref, out_ref, feat_ref, s_ref, ss_ref):
    ph = pl.program_id(0)
    i = pl.program_id(1)

    @pl.when(ph == 0)
    def _():
        xb = x_ref[...]
        aggb = p_ref[0] + p_ref[1]
        h0 = jnp.maximum(
            jnp.dot(xb, w0_ref[...], preferred_element_type=jnp.float32)
            + b0_ref[...], 0.0)
        h1 = jnp.maximum(
            jnp.dot(aggb, w1_ref[...], preferred_element_type=jnp.float32)
            + b1_ref[...], 0.0)
        f = h0 + h1
        feat_ref[pl.ds(i * BLK, BLK), :] = f
        sb = jnp.sum(f, axis=0, keepdims=True)
        ssb = jnp.sum(f * f, axis=0, keepdims=True)

        @pl.when(i == 0)
        def _():
            s_ref[...] = sb
            ss_ref[...] = ssb

        @pl.when(i != 0)
        def _():
            s_ref[...] += sb
            ss_ref[...] += ssb

    @pl.when(ph == 1)
    def _():
        mean = s_ref[...] / N
        var = ss_ref[...] / N - mean * mean
        scale = lax.rsqrt(var + 1e-9) * g_ref[...]
        out_ref[...] = (feat_ref[pl.ds(i * BLK, BLK), :] * scale
                        + (bt_ref[...] - mean * scale))


def kernel(x, edge_index, edge_weight, W0, W1, b0, b1, gamma, beta):
    packed = jnp.bitwise_or(jnp.left_shift(edge_index[0], 16), edge_index[1])

    part = _build_sc_spmm()(x, packed, edge_weight)

    out = pl.pallas_call(
        _tc_fused_kernel,
        grid=(2, N // BLK),
        in_specs=[
            pl.BlockSpec((BLK, D), lambda p, i: (i * (1 - p), 0)),
            pl.BlockSpec((NC, BLK, D), lambda p, i: (0, i * (1 - p), 0)),
            pl.BlockSpec((D, D), lambda p, i: (0, 0)),
            pl.BlockSpec((D, D), lambda p, i: (0, 0)),
            pl.BlockSpec((1, D), lambda p, i: (0, 0)),
            pl.BlockSpec((1, D), lambda p, i: (0, 0)),
            pl.BlockSpec((1, D), lambda p, i: (0, 0)),
            pl.BlockSpec((1, D), lambda p, i: (0, 0)),
        ],
        out_specs=pl.BlockSpec((BLK, D), lambda p, i: (i * p, 0)),
        out_shape=jax.ShapeDtypeStruct((N, D), jnp.float32),
        scratch_shapes=[
            pltpu.VMEM((N, D), jnp.float32),
            pltpu.VMEM((1, D), jnp.float32),
            pltpu.VMEM((1, D), jnp.float32),
        ],
    )(x, part, W0, W1, b0[None, :], b1[None, :],
      gamma[None, :], beta[None, :])
    return out


# speedup vs baseline: 10.1149x; 1.0004x over previous
import functools

import jax
import jax.numpy as jnp
from jax import lax
from jax.experimental import pallas as pl
from jax.experimental.pallas import tpu as pltpu
from jax.experimental.pallas import tpu_sc as plsc

N = 10000
E = 320000
D = 128

NC = 2
NS = 16
NW = NC * NS
EPT = E // NW
CH = 80
NCHUNK = EPT // CH
ZR = 624
ZR_LAST = N - (NS - 1) * ZR


@functools.lru_cache(maxsize=1)
def _build_sc_spmm():
    mesh = plsc.VectorSubcoreMesh(core_axis_name="c", subcore_axis_name="s")

    @functools.partial(
        pl.kernel,
        out_type=jax.ShapeDtypeStruct((NC, N, D), jnp.float32),
        mesh=mesh,
        scratch_types=[
            pltpu.VMEM((EPT,), jnp.int32),
            pltpu.VMEM((EPT,), jnp.float32),
            pltpu.VMEM((CH, D), jnp.float32),
            pltpu.VMEM((CH, D), jnp.float32),
            pltpu.VMEM((CH, D), jnp.float32),
            pltpu.VMEM_SHARED((N, D), jnp.float32),
            pltpu.SemaphoreType.DMA,
            pltpu.SemaphoreType.DMA,
            pltpu.SemaphoreType.DMA,
            pltpu.SemaphoreType.DMA,
            pltpu.SemaphoreType.DMA,
            pltpu.SemaphoreType.DMA,
        ],
    )
    def sc_spmm(x_hbm, packed_hbm, w_hbm, out_hbm,
                pall, wbuf, b0, b1, b2, aggbuf,
                gsem0, gsem1, gsem2, ssem0, ssem1, ssem2):
        c = lax.axis_index("c")
        s = lax.axis_index("s")
        wid = s * NC + c
        ebase = wid * EPT
        sbase = pl.multiple_of(s * ZR, 8)

        pltpu.sync_copy(packed_hbm.at[pl.ds(ebase, EPT)], pall)
        pltpu.sync_copy(w_hbm.at[pl.ds(ebase, EPT)], wbuf)

        zv = jnp.zeros((16,), jnp.float32)

        def zero_body(i, carry):
            for j in range(D // 16):
                b0[i, pl.ds(j * 16, 16)] = zv
            return carry

        lax.fori_loop(0, CH, zero_body, 0)
        for m in range(ZR // CH):
            pltpu.async_copy(b0, aggbuf.at[pl.ds(sbase + m * CH, CH)], gsem0)
        for m in range(ZR // CH):
            pltpu.make_async_copy(
                b0, aggbuf.at[pl.ds(sbase + m * CH, CH)], gsem0).wait()
        ZT = ZR - (ZR // CH) * CH

        @pl.when(s < NS - 1)
        def _():
            pltpu.sync_copy(b0.at[pl.ds(0, ZT)],
                            aggbuf.at[pl.ds(sbase + (ZR // CH) * CH, ZT)])

        @pl.when(s == NS - 1)
        def _():
            pltpu.sync_copy(b0,
                            aggbuf.at[pl.ds((NS - 1) * ZR + (ZR // CH) * CH,
                                            CH)])
        plsc.subcore_barrier()

        NG = CH // 16

        def src_idx(k, g):
            pk = pall[pl.ds(k * CH + g * 16, 16)]
            return jnp.bitwise_and(pk, 0xFFFF)

        def dst_idx(k, g):
            pk = pall[pl.ds(k * CH + g * 16, 16)]
            return jnp.right_shift(pk, 16)

        def launch_gather(k, buf, sem):
            for g in range(NG):
                pltpu.async_copy(x_hbm.at[src_idx(k, g)],
                                 buf.at[pl.ds(g * 16, 16)], sem)

        def wait_gather(k, buf, sem):
            for g in range(NG):
                pltpu.make_async_copy(x_hbm.at[src_idx(k, g)],
                                      buf.at[pl.ds(g * 16, 16)], sem).wait()

        def launch_scatter(k, buf, sem):
            for g in range(NG):
                pltpu.async_copy(buf.at[pl.ds(g * 16, 16)],
                                 aggbuf.at[dst_idx(k, g)], sem, add=True)

        def wait_scatter(k, buf, sem):
            for g in range(NG):
                pltpu.make_async_copy(buf.at[pl.ds(g * 16, 16)],
                                      aggbuf.at[dst_idx(k, g)], sem).wait()

        def process_chunk(rows_ref, k, ssem):
            def group_body(g, carry2):
                wv = wbuf[pl.ds(k * CH + g * 16, 16)]
                for l in range(16):
                    w = wv[l]
                    for j in range(D // 16):
                        sl = pl.ds(j * 16, 16)
                        rows_ref[g * 16 + l, sl] = \
                            rows_ref[g * 16 + l, sl] * w
                pltpu.async_copy(rows_ref.at[pl.ds(g * 16, 16)],
                                 aggbuf.at[dst_idx(k, g)],
                                 ssem, add=True)
                return carry2

            lax.fori_loop(0, CH // 16, group_body, 0)

        bufs = (b0, b1, b2)
        gsems = (gsem0, gsem1, gsem2)
        ssems = (ssem0, ssem1, ssem2)

        launch_gather(0, b0, gsem0)

        def chunk_body(k, carry):
            m = lax.rem(k, 3)
            for b in range(3):
                nb = (b + 1) % 3

                @pl.when(m == b)
                def _(b=b, nb=nb):
                    wait_gather(k, bufs[b], gsems[b])

                    @pl.when(k + 1 < NCHUNK)
                    def _():
                        @pl.when(k >= 2)
                        def _():
                            wait_scatter(k - 2, bufs[nb], ssems[nb])

                        launch_gather(k + 1, bufs[nb], gsems[nb])

                    process_chunk(bufs[b], k, ssems[b])

            return carry

        lax.fori_loop(0, NCHUNK, chunk_body, 0)
        for kk in (NCHUNK - 3, NCHUNK - 2, NCHUNK - 1):
            wait_scatter(kk, bufs[kk % 3], ssems[kk % 3])
        plsc.subcore_barrier()

        @pl.when(s < NS - 1)
        def _():
            pltpu.sync_copy(aggbuf.at[pl.ds(sbase, ZR)],
                            out_hbm.at[c, pl.ds(sbase, ZR)])

        @pl.when(s == NS - 1)
        def _():
            pltpu.sync_copy(aggbuf.at[pl.ds((NS - 1) * ZR, ZR_LAST)],
                            out_hbm.at[c, pl.ds((NS - 1) * ZR, ZR_LAST)])

    return sc_spmm


BLK = 1000


def _tc_fused_kernel(x_ref, p_ref, w0_ref, w1_ref, b0_ref, b1_ref,
                     g_ref, bt_ref, out_ref, feat_ref, s_ref, ss_ref):
    ph = pl.program_id(0)
    i = pl.program_id(1)

    @pl.when(ph == 0)
    def _():
        xb = x_ref[...]
        aggb = p_ref[0] + p_ref[1]
        h0 = jnp.maximum(
            jnp.dot(xb, w0_ref[...], preferred_element_type=jnp.float32)
            + b0_ref[...], 0.0)
        h1 = jnp.maximum(
            jnp.dot(aggb, w1_ref[...], preferred_element_type=jnp.float32)
            + b1_ref[...], 0.0)
        f = h0 + h1
        feat_ref[pl.ds(i * BLK, BLK), :] = f
        sb = jnp.sum(f, axis=0, keepdims=True)
        ssb = jnp.sum(f * f, axis=0, keepdims=True)

        @pl.when(i == 0)
        def _():
            s_ref[...] = sb
            ss_ref[...] = ssb

        @pl.when(i != 0)
        def _():
            s_ref[...] += sb
            ss_ref[...] += ssb

    @pl.when(ph == 1)
    def _():
        mean = s_ref[...] / N
        var = ss_ref[...] / N - mean * mean
        scale = lax.rsqrt(var + 1e-9) * g_ref[...]
        out_ref[...] = (feat_ref[pl.ds(i * BLK, BLK), :] * scale
                        + (bt_ref[...] - mean * scale))


def kernel(x, edge_index, edge_weight, W0, W1, b0, b1, gamma, beta):
    packed = jnp.bitwise_or(jnp.left_shift(edge_index[0], 16), edge_index[1])

    part = _build_sc_spmm()(x, packed, edge_weight)

    out = pl.pallas_call(
        _tc_fused_kernel,
        grid=(2, N // BLK),
        in_specs=[
            pl.BlockSpec((BLK, D), lambda p, i: (i * (1 - p), 0)),
            pl.BlockSpec((NC, BLK, D), lambda p, i: (0, i * (1 - p), 0)),
            pl.BlockSpec((D, D), lambda p, i: (0, 0)),
            pl.BlockSpec((D, D), lambda p, i: (0, 0)),
            pl.BlockSpec((1, D), lambda p, i: (0, 0)),
            pl.BlockSpec((1, D), lambda p, i: (0, 0)),
            pl.BlockSpec((1, D), lambda p, i: (0, 0)),
            pl.BlockSpec((1, D), lambda p, i: (0, 0)),
        ],
        out_specs=pl.BlockSpec((BLK, D), lambda p, i: (i * p, 0)),
        out_shape=jax.ShapeDtypeStruct((N, D), jnp.float32),
        scratch_shapes=[
            pltpu.VMEM((N, D), jnp.float32),
            pltpu.VMEM((1, D), jnp.float32),
            pltpu.VMEM((1, D), jnp.float32),
        ],
    )(x, part, W0, W1, b0[None, :], b1[None, :],
      gamma[None, :], beta[None, :])
    return out


# speedup vs baseline: 12.9399x; 1.2793x over previous
import functools

import jax
import jax.numpy as jnp
from jax import lax
from jax.experimental import pallas as pl
from jax.experimental.pallas import tpu as pltpu
from jax.experimental.pallas import tpu_sc as plsc

N = 10000
E = 320000
D = 128

NC = 2
NS = 16
NW = NC * NS
EPT = E // NW
CH = 80
NCHUNK = EPT // CH
ZR = 624
ZR_LAST = N - (NS - 1) * ZR


@functools.lru_cache(maxsize=1)
def _build_sc_spmm():
    mesh = plsc.VectorSubcoreMesh(core_axis_name="c", subcore_axis_name="s")

    @functools.partial(
        pl.kernel,
        out_type=jax.ShapeDtypeStruct((NC, N, D), jnp.float32),
        mesh=mesh,
        scratch_types=[
            pltpu.VMEM((EPT,), jnp.int32),
            pltpu.VMEM((EPT,), jnp.float32),
            pltpu.VMEM((CH, D), jnp.float32),
            pltpu.VMEM((CH, D), jnp.float32),
            pltpu.VMEM((CH, D), jnp.float32),
            pltpu.VMEM_SHARED((N, D), jnp.float32),
            pltpu.SemaphoreType.DMA,
            pltpu.SemaphoreType.DMA,
            pltpu.SemaphoreType.DMA,
            pltpu.SemaphoreType.DMA,
            pltpu.SemaphoreType.DMA,
            pltpu.SemaphoreType.DMA,
        ],
    )
    def sc_spmm(x_hbm, packed_hbm, w_hbm, out_hbm,
                pall, wbuf, b0, b1, b2, aggbuf,
                gsem0, gsem1, gsem2, ssem0, ssem1, ssem2):
        c = lax.axis_index("c")
        s = lax.axis_index("s")
        wid = s * NC + c
        ebase = wid * EPT
        sbase = pl.multiple_of(s * ZR, 8)

        pltpu.sync_copy(packed_hbm.at[pl.ds(ebase, EPT)], pall)
        pltpu.sync_copy(w_hbm.at[pl.ds(ebase, EPT)], wbuf)

        zv = jnp.zeros((16,), jnp.float32)

        def zero_body(i, carry):
            for j in range(D // 16):
                b0[i, pl.ds(j * 16, 16)] = zv
            return carry

        lax.fori_loop(0, CH, zero_body, 0)
        for m in range(ZR // CH):
            pltpu.async_copy(b0, aggbuf.at[pl.ds(sbase + m * CH, CH)], gsem0)
        for m in range(ZR // CH):
            pltpu.make_async_copy(
                b0, aggbuf.at[pl.ds(sbase + m * CH, CH)], gsem0).wait()
        ZT = ZR - (ZR // CH) * CH

        @pl.when(s < NS - 1)
        def _():
            pltpu.sync_copy(b0.at[pl.ds(0, ZT)],
                            aggbuf.at[pl.ds(sbase + (ZR // CH) * CH, ZT)])

        @pl.when(s == NS - 1)
        def _():
            pltpu.sync_copy(b0,
                            aggbuf.at[pl.ds((NS - 1) * ZR + (ZR // CH) * CH,
                                            CH)])
        plsc.subcore_barrier()

        NG = CH // 16

        def src_idx(k, g):
            pk = pall[pl.ds(k * CH + g * 16, 16)]
            return jnp.bitwise_and(pk, 0xFFFF)

        def dst_idx(k, g):
            pk = pall[pl.ds(k * CH + g * 16, 16)]
            return jnp.right_shift(pk, 16)

        def launch_gather(k, buf, sem):
            for g in range(NG):
                pltpu.async_copy(x_hbm.at[src_idx(k, g)],
                                 buf.at[pl.ds(g * 16, 16)], sem)

        def wait_gather(k, buf, sem):
            for g in range(NG):
                pltpu.make_async_copy(x_hbm.at[src_idx(k, g)],
                                      buf.at[pl.ds(g * 16, 16)], sem).wait()

        def launch_scatter(k, buf, sem):
            for g in range(NG):
                pltpu.async_copy(buf.at[pl.ds(g * 16, 16)],
                                 aggbuf.at[dst_idx(k, g)], sem, add=True)

        def wait_scatter(k, buf, sem):
            for g in range(NG):
                pltpu.make_async_copy(buf.at[pl.ds(g * 16, 16)],
                                      aggbuf.at[dst_idx(k, g)], sem).wait()

        def process_chunk(rows_ref, k, ssem):
            def group_body(g, carry2):
                wv = wbuf[pl.ds(k * CH + g * 16, 16)]
                for l in range(16):
                    w = wv[l]
                    for j in range(D // 16):
                        sl = pl.ds(j * 16, 16)
                        rows_ref[g * 16 + l, sl] = \
                            rows_ref[g * 16 + l, sl] * w
                pltpu.async_copy(rows_ref.at[pl.ds(g * 16, 16)],
                                 aggbuf.at[dst_idx(k, g)],
                                 ssem, add=True)
                return carry2

            lax.fori_loop(0, CH // 16, group_body, 0)

        bufs = (b0, b1, b2)
        gsems = (gsem0, gsem1, gsem2)
        ssems = (ssem0, ssem1, ssem2)

        launch_gather(0, b0, gsem0)
        launch_gather(1, b1, gsem1)

        def chunk_body(k, carry):
            m = lax.rem(k, 3)
            for b in range(3):
                pb = (b + 2) % 3

                @pl.when(m == b)
                def _(b=b, pb=pb):
                    wait_gather(k, bufs[b], gsems[b])

                    @pl.when(k >= 1)
                    def _():
                        wait_scatter(k - 1, bufs[pb], ssems[pb])

                    @pl.when(k + 2 < NCHUNK)
                    def _():
                        launch_gather(k + 2, bufs[pb], gsems[pb])

                    process_chunk(bufs[b], k, ssems[b])

            return carry

        lax.fori_loop(0, NCHUNK, chunk_body, 0)
        wait_scatter(NCHUNK - 1, bufs[(NCHUNK - 1) % 3],
                     ssems[(NCHUNK - 1) % 3])
        plsc.subcore_barrier()

        @pl.when(s < NS - 1)
        def _():
            pltpu.sync_copy(aggbuf.at[pl.ds(sbase, ZR)],
                            out_hbm.at[c, pl.ds(sbase, ZR)])

        @pl.when(s == NS - 1)
        def _():
            pltpu.sync_copy(aggbuf.at[pl.ds((NS - 1) * ZR, ZR_LAST)],
                            out_hbm.at[c, pl.ds((NS - 1) * ZR, ZR_LAST)])

    return sc_spmm


BLK = 1000


def _tc_fused_kernel(x_ref, p_ref, w0_ref, w1_ref, b0_ref, b1_ref,
                     g_ref, bt_ref, out_ref, feat_ref, s_ref, ss_ref):
    ph = pl.program_id(0)
    i = pl.program_id(1)

    @pl.when(ph == 0)
    def _():
        xb = x_ref[...]
        aggb = p_ref[0] + p_ref[1]
        h0 = jnp.maximum(
            jnp.dot(xb, w0_ref[...], preferred_element_type=jnp.float32)
            + b0_ref[...], 0.0)
        h1 = jnp.maximum(
            jnp.dot(aggb, w1_ref[...], preferred_element_type=jnp.float32)
            + b1_ref[...], 0.0)
        f = h0 + h1
        feat_ref[pl.ds(i * BLK, BLK), :] = f
        sb = jnp.sum(f, axis=0, keepdims=True)
        ssb = jnp.sum(f * f, axis=0, keepdims=True)

        @pl.when(i == 0)
        def _():
            s_ref[...] = sb
            ss_ref[...] = ssb

        @pl.when(i != 0)
        def _():
            s_ref[...] += sb
            ss_ref[...] += ssb

    @pl.when(ph == 1)
    def _():
        mean = s_ref[...] / N
        var = ss_ref[...] / N - mean * mean
        scale = lax.rsqrt(var + 1e-9) * g_ref[...]
        out_ref[...] = (feat_ref[pl.ds(i * BLK, BLK), :] * scale
                        + (bt_ref[...] - mean * scale))


def kernel(x, edge_index, edge_weight, W0, W1, b0, b1, gamma, beta):
    packed = jnp.bitwise_or(jnp.left_shift(edge_index[0], 16), edge_index[1])

    part = _build_sc_spmm()(x, packed, edge_weight)

    out = pl.pallas_call(
        _tc_fused_kernel,
        grid=(2, N // BLK),
        in_specs=[
            pl.BlockSpec((BLK, D), lambda p, i: (i * (1 - p), 0)),
            pl.BlockSpec((NC, BLK, D), lambda p, i: (0, i * (1 - p), 0)),
            pl.BlockSpec((D, D), lambda p, i: (0, 0)),
            pl.BlockSpec((D, D), lambda p, i: (0, 0)),
            pl.BlockSpec((1, D), lambda p, i: (0, 0)),
            pl.BlockSpec((1, D), lambda p, i: (0, 0)),
            pl.BlockSpec((1, D), lambda p, i: (0, 0)),
            pl.BlockSpec((1, D), lambda p, i: (0, 0)),
        ],
        out_specs=pl.BlockSpec((BLK, D), lambda p, i: (i * p, 0)),
        out_shape=jax.ShapeDtypeStruct((N, D), jnp.float32),
        scratch_shapes=[
            pltpu.VMEM((N, D), jnp.float32),
            pltpu.VMEM((1, D), jnp.float32),
            pltpu.VMEM((1, D), jnp.float32),
        ],
    )(x, part, W0, W1, b0[None, :], b1[None, :],
      gamma[None, :], beta[None, :])
    return out
